# Initial kernel scaffold; baseline (speedup 1.0000x reference)
import jax
import jax.numpy as jnp
from jax.experimental import pallas as pl


def kernel(x_user, x_item, edge_index_ui, edge_index_iu, W_l_ui, W_r_ui, W_l_iu, W_r_iu):
    raise NotImplementedError("write your pallas kernel here")


# speedup vs baseline: 1.1839x; 1.1839x over previous
import functools

import jax
import jax.numpy as jnp
from jax import lax
from jax.experimental import pallas as pl
from jax.experimental.pallas import tpu as pltpu
from jax.experimental.pallas import tpu_sc as plsc

N_NODES = 10000
NP = 10240
D = 256
DH = 128
E = 160000

NS = 16
RPT = NP // NS
EB = 2000
NB = E // EB
GC = 64
CNTW = 16
HPT = RPT // 2
RA = 256
RB = HPT - RA


def _sc_segment_means(xu0, xu1, xi0, xi1, sui, dui, siu, diu):
    mesh = plsc.VectorSubcoreMesh(core_axis_name="c", subcore_axis_name="s")
    f32 = jnp.float32
    i32 = jnp.int32

    @functools.partial(
        pl.kernel,
        out_type=[
            jax.ShapeDtypeStruct((NP, DH), f32),
            jax.ShapeDtypeStruct((NP, DH), f32),
            jax.ShapeDtypeStruct((NP, DH), f32),
            jax.ShapeDtypeStruct((NP, DH), f32),
        ],
        mesh=mesh,
        compiler_params=pltpu.CompilerParams(needs_layout_passes=False),
    )
    def k(xu0_h, xu1_h, xi0_h, xi1_h, sui_h, dui_h, siu_h, diu_h,
          mean_i0, mean_i1, mean_u0, mean_u1):
        pl.run_scoped(
            functools.partial(
                _tile_body, xu0_h, xu1_h, xi0_h, xi1_h,
                sui_h, dui_h, siu_h, diu_h,
                mean_i0, mean_i1, mean_u0, mean_u1),
            pltpu.VMEM((RA, DH), f32),
            pltpu.VMEM((RB, DH), f32),
            pltpu.VMEM((RA, CNTW), f32),
            pltpu.VMEM((RB, CNTW), f32),
            pltpu.VMEM((GC, DH), f32),
            pltpu.VMEM((EB + 16,), i32),
            pltpu.VMEM((EB + 16,), i32),
            pltpu.VMEM((GC,), i32),
        )

    def _tile_body(xu0_h, xu1_h, xi0_h, xi1_h, sui_h, dui_h, siu_h, diu_h,
                   mean_i0, mean_i1, mean_u0, mean_u1,
                   acc_a, acc_b, cnt_a, cnt_b, stage, sstage, dstage, gidx):
        c = lax.axis_index("c")
        s = lax.axis_index("s")
        row0 = s * RPT

        one16 = jnp.full((16,), 1.0, f32)
        zero16 = jnp.zeros((16,), f32)
        zi16 = jnp.zeros((16,), i32)

        for rel in range(2):
          x0_h, x1_h = (xu0_h, xu1_h) if rel == 0 else (xi0_h, xi1_h)
          si_h, di_h = (sui_h, dui_h) if rel == 0 else (siu_h, diu_h)
          o0, o1 = (mean_i0, mean_i1) if rel == 0 else (mean_u0, mean_u1)
          for half in range(2):
            rowb = row0 + half * HPT

            def zero_row_a(r, _):
                for kk in range(DH // 16):
                    acc_a[r, pl.ds(kk * 16, 16)] = zero16
                cnt_a[r] = zero16
                return 0

            def zero_row_b(r, _):
                for kk in range(DH // 16):
                    acc_b[r, pl.ds(kk * 16, 16)] = zero16
                cnt_b[r] = zero16
                return 0

            lax.fori_loop(0, RA, zero_row_a, 0)
            lax.fori_loop(0, RB, zero_row_b, 0)

            def block_body(b, _):
                pltpu.sync_copy(si_h.at[pl.ds(b * EB, EB + 16)], sstage)
                pltpu.sync_copy(di_h.at[pl.ds(b * EB, EB + 16)], dstage)

                def scan_step(v, p):
                    d16 = dstage[pl.ds(v * 16, 16)]
                    s16 = sstage[pl.ds(v * 16, 16)]
                    m = (d16 - rowb).astype(jnp.uint32) < jnp.uint32(HPT)
                    mi = jnp.where(m, jnp.int32(1), jnp.int32(0))
                    pos = plsc.cumsum(mi)
                    idx = jnp.where(m, p + pos - 1, jnp.int32(EB))
                    plsc.store_scatter(sstage, [idx], s16)
                    plsc.store_scatter(dstage, [idx], d16)
                    return p + pos[15]

                p = lax.fori_loop(0, EB // 16, scan_step, jnp.int32(0))

                def chunk_body(g, _):
                    for kk in range(GC // 16):
                        gidx[pl.ds(kk * 16, 16)] = (
                            sstage[pl.ds(g * GC + kk * 16, 16)])

                    @pl.when(c == 0)
                    def _():
                        pltpu.sync_copy(x0_h.at[gidx], stage)

                    @pl.when(c == 1)
                    def _():
                        pltpu.sync_copy(x1_h.at[gidx], stage)

                    n = jnp.minimum(p - g * GC, GC)

                    def edge_body(e, _):
                        dloc = dstage[pl.ds(g * GC + e, 16)][0] - rowb

                        @pl.when(dloc < RA)
                        def _():
                            for kk in range(DH // 16):
                                plsc.addupdate(
                                    acc_a.at[dloc, pl.ds(kk * 16, 16)],
                                    stage[e, pl.ds(kk * 16, 16)])
                            plsc.addupdate(cnt_a.at[dloc], one16)

                        @pl.when(dloc >= RA)
                        def _():
                            for kk in range(DH // 16):
                                plsc.addupdate(
                                    acc_b.at[dloc - RA, pl.ds(kk * 16, 16)],
                                    stage[e, pl.ds(kk * 16, 16)])
                            plsc.addupdate(cnt_b.at[dloc - RA], one16)
                        return 0

                    lax.fori_loop(0, n, edge_body, 0)
                    return 0

                lax.fori_loop(0, (p + GC - 1) // GC, chunk_body, 0)
                return 0

            lax.fori_loop(0, NB, block_body, 0)

            for cc in range(HPT // GC):
                part_acc = acc_a if cc * GC < RA else acc_b
                part_cnt = cnt_a if cc * GC < RA else cnt_b
                rbase = cc * GC if cc * GC < RA else cc * GC - RA

                def scale_row(r, _, part_acc=part_acc, part_cnt=part_cnt,
                              rbase=rbase):
                    rr = rbase + r
                    c16 = part_cnt[rr]
                    inv = 1.0 / jnp.maximum(c16, 1.0)
                    for kk in range(DH // 16):
                        stage[r, pl.ds(kk * 16, 16)] = (
                            part_acc[rr, pl.ds(kk * 16, 16)] * inv)
                    return 0

                lax.fori_loop(0, GC, scale_row, 0)
                orow = rowb + cc * GC

                @pl.when(c == 0)
                def _():
                    pltpu.sync_copy(stage, o0.at[pl.ds(orow, GC)])

                @pl.when(c == 1)
                def _():
                    pltpu.sync_copy(stage, o1.at[pl.ds(orow, GC)])

        return None

    _ = _tile_body
    return k(xu0, xu1, xi0, xi1, sui, dui, siu, diu)


ROWS_BLK = 400


def _tc_body(m0_ref, m1_ref, xd_ref, wl_ref, wr_ref, out_ref):
    out_ref[...] = (
        jnp.dot(m0_ref[...], wl_ref[0:DH, :], preferred_element_type=jnp.float32)
        + jnp.dot(m1_ref[...], wl_ref[DH:D, :], preferred_element_type=jnp.float32)
        + jnp.dot(xd_ref[...], wr_ref[...], preferred_element_type=jnp.float32)
    )


def _tc_sage_update(m0, m1, x_dst, W_l, W_r):
    grid = (N_NODES // ROWS_BLK,)
    return pl.pallas_call(
        _tc_body,
        grid=grid,
        in_specs=[
            pl.BlockSpec((ROWS_BLK, DH), lambda b: (b, 0)),
            pl.BlockSpec((ROWS_BLK, DH), lambda b: (b, 0)),
            pl.BlockSpec((ROWS_BLK, D), lambda b: (b, 0)),
            pl.BlockSpec((D, D), lambda b: (0, 0)),
            pl.BlockSpec((D, D), lambda b: (0, 0)),
        ],
        out_specs=pl.BlockSpec((ROWS_BLK, D), lambda b: (b, 0)),
        out_shape=jax.ShapeDtypeStruct((N_NODES, D), jnp.float32),
    )(m0, m1, x_dst, W_l, W_r)


def kernel(x_user, x_item, edge_index_ui, edge_index_iu, W_l_ui, W_r_ui, W_l_iu, W_r_iu):
    xu0 = x_user[:, :DH]
    xu1 = x_user[:, DH:]
    xi0 = x_item[:, :DH]
    xi1 = x_item[:, DH:]
    pad = jnp.zeros((16,), jnp.int32)
    sui = jnp.concatenate([edge_index_ui[0].astype(jnp.int32), pad])
    dui = jnp.concatenate([edge_index_ui[1].astype(jnp.int32), pad])
    siu = jnp.concatenate([edge_index_iu[0].astype(jnp.int32), pad])
    diu = jnp.concatenate([edge_index_iu[1].astype(jnp.int32), pad])

    mean_i0, mean_i1, mean_u0, mean_u1 = _sc_segment_means(
        xu0, xu1, xi0, xi1, sui, dui, siu, diu)

    out_item = _tc_sage_update(mean_i0, mean_i1, x_item, W_l_ui, W_r_ui)
    out_user = _tc_sage_update(mean_u0, mean_u1, x_user, W_l_iu, W_r_iu)
    return (out_user, out_item)


# speedup vs baseline: 1.4202x; 1.1996x over previous
import functools

import jax
import jax.numpy as jnp
from jax import lax
from jax.experimental import pallas as pl
from jax.experimental.pallas import tpu as pltpu
from jax.experimental.pallas import tpu_sc as plsc

N_NODES = 10000
NP = 10240
D = 256
DH = 128
E = 160000

NS = 16
RPT = NP // NS
EB = 4000
NB = E // EB
GC = 128
SCR = 64
CNTW = 16
HPT = RPT // 2
RA = 256
RB = HPT - RA


def _sc_segment_means(xu0, xu1, xi0, xi1, sui, dui, siu, diu):
    mesh = plsc.VectorSubcoreMesh(core_axis_name="c", subcore_axis_name="s")
    f32 = jnp.float32
    i32 = jnp.int32

    @functools.partial(
        pl.kernel,
        out_type=[
            jax.ShapeDtypeStruct((NP, DH), f32),
            jax.ShapeDtypeStruct((NP, DH), f32),
            jax.ShapeDtypeStruct((NP, DH), f32),
            jax.ShapeDtypeStruct((NP, DH), f32),
        ],
        mesh=mesh,
        compiler_params=pltpu.CompilerParams(needs_layout_passes=False),
    )
    def k(xu0_h, xu1_h, xi0_h, xi1_h, sui_h, dui_h, siu_h, diu_h,
          mean_i0, mean_i1, mean_u0, mean_u1):
        pl.run_scoped(
            functools.partial(
                _tile_body, xu0_h, xu1_h, xi0_h, xi1_h,
                sui_h, dui_h, siu_h, diu_h,
                mean_i0, mean_i1, mean_u0, mean_u1),
            pltpu.VMEM((RA, DH), f32),
            pltpu.VMEM((RB, DH), f32),
            pltpu.VMEM((RA, CNTW), f32),
            pltpu.VMEM((RB, CNTW), f32),
            pltpu.VMEM((GC, DH), f32),
            pltpu.VMEM((EB + 16,), i32),
            pltpu.VMEM((EB + 16,), i32),
            pltpu.VMEM((GC,), i32),
        )

    def _tile_body(xu0_h, xu1_h, xi0_h, xi1_h, sui_h, dui_h, siu_h, diu_h,
                   mean_i0, mean_i1, mean_u0, mean_u1,
                   acc_a, acc_b, cnt_a, cnt_b, stage, sstage, dstage, gidx):
        c = lax.axis_index("c")
        s = lax.axis_index("s")
        row0 = s * RPT

        one16 = jnp.full((16,), 1.0, f32)
        zero16 = jnp.zeros((16,), f32)
        zi16 = jnp.zeros((16,), i32)

        for rel in range(2):
          x0_h, x1_h = (xu0_h, xu1_h) if rel == 0 else (xi0_h, xi1_h)
          si_h, di_h = (sui_h, dui_h) if rel == 0 else (siu_h, diu_h)
          o0, o1 = (mean_i0, mean_i1) if rel == 0 else (mean_u0, mean_u1)
          for half in range(2):
            rowb = row0 + half * HPT

            def zero_row_a(r, _):
                for kk in range(DH // 16):
                    acc_a[r, pl.ds(kk * 16, 16)] = zero16
                cnt_a[r] = zero16
                return 0

            def zero_row_b(r, _):
                for kk in range(DH // 16):
                    acc_b[r, pl.ds(kk * 16, 16)] = zero16
                cnt_b[r] = zero16
                return 0

            lax.fori_loop(0, RA, zero_row_a, 0)
            lax.fori_loop(0, RB, zero_row_b, 0)

            def block_body(b, _):
                pltpu.sync_copy(si_h.at[pl.ds(b * EB, EB + 16)], sstage)
                pltpu.sync_copy(di_h.at[pl.ds(b * EB, EB + 16)], dstage)

                def scan_step(v, p):
                    d16 = dstage[pl.ds(v * 16, 16)]
                    s16 = sstage[pl.ds(v * 16, 16)]
                    m = (d16 - rowb).astype(jnp.uint32) < jnp.uint32(HPT)
                    mi = jnp.where(m, jnp.int32(1), jnp.int32(0))
                    pos = plsc.cumsum(mi)
                    idx = jnp.where(m, p + pos - 1, jnp.int32(EB))
                    plsc.store_scatter(sstage, [idx], s16)
                    plsc.store_scatter(dstage, [idx], d16)
                    return p + pos[15]

                p = lax.fori_loop(0, EB // 16, scan_step, jnp.int32(0))

                def chunk_body(g, _):
                    for kk in range(GC // 16):
                        gidx[pl.ds(kk * 16, 16)] = (
                            sstage[pl.ds(g * GC + kk * 16, 16)])

                    @pl.when(c == 0)
                    def _():
                        pltpu.sync_copy(x0_h.at[gidx], stage)

                    @pl.when(c == 1)
                    def _():
                        pltpu.sync_copy(x1_h.at[gidx], stage)

                    n = jnp.minimum(p - g * GC, GC)

                    def edge_body(e, _):
                        dloc = dstage[pl.ds(g * GC + e, 16)][0] - rowb

                        @pl.when(dloc < RA)
                        def _():
                            for kk in range(DH // 16):
                                plsc.addupdate(
                                    acc_a.at[dloc, pl.ds(kk * 16, 16)],
                                    stage[e, pl.ds(kk * 16, 16)])
                            plsc.addupdate(cnt_a.at[dloc], one16)

                        @pl.when(dloc >= RA)
                        def _():
                            for kk in range(DH // 16):
                                plsc.addupdate(
                                    acc_b.at[dloc - RA, pl.ds(kk * 16, 16)],
                                    stage[e, pl.ds(kk * 16, 16)])
                            plsc.addupdate(cnt_b.at[dloc - RA], one16)
                        return 0

                    lax.fori_loop(0, n, edge_body, 0)
                    return 0

                lax.fori_loop(0, (p + GC - 1) // GC, chunk_body, 0)
                return 0

            lax.fori_loop(0, NB, block_body, 0)

            for cc in range(HPT // SCR):
                part_acc = acc_a if cc * SCR < RA else acc_b
                part_cnt = cnt_a if cc * SCR < RA else cnt_b
                rbase = cc * SCR if cc * SCR < RA else cc * SCR - RA

                def scale_row(r, _, part_acc=part_acc, part_cnt=part_cnt,
                              rbase=rbase):
                    rr = rbase + r
                    c16 = part_cnt[rr]
                    inv = 1.0 / jnp.maximum(c16, 1.0)
                    for kk in range(DH // 16):
                        stage[r, pl.ds(kk * 16, 16)] = (
                            part_acc[rr, pl.ds(kk * 16, 16)] * inv)
                    return 0

                lax.fori_loop(0, SCR, scale_row, 0)
                orow = rowb + cc * SCR

                @pl.when(c == 0)
                def _():
                    pltpu.sync_copy(stage.at[pl.ds(0, SCR)],
                                    o0.at[pl.ds(orow, SCR)])

                @pl.when(c == 1)
                def _():
                    pltpu.sync_copy(stage.at[pl.ds(0, SCR)],
                                    o1.at[pl.ds(orow, SCR)])

        return None

    _ = _tile_body
    return k(xu0, xu1, xi0, xi1, sui, dui, siu, diu)


ROWS_BLK = 400


def _tc_body(m0_ref, m1_ref, xd_ref, wl_ref, wr_ref, out_ref):
    out_ref[...] = (
        jnp.dot(m0_ref[...], wl_ref[0:DH, :], preferred_element_type=jnp.float32)
        + jnp.dot(m1_ref[...], wl_ref[DH:D, :], preferred_element_type=jnp.float32)
        + jnp.dot(xd_ref[...], wr_ref[...], preferred_element_type=jnp.float32)
    )


def _tc_sage_update(m0, m1, x_dst, W_l, W_r):
    grid = (N_NODES // ROWS_BLK,)
    return pl.pallas_call(
        _tc_body,
        grid=grid,
        in_specs=[
            pl.BlockSpec((ROWS_BLK, DH), lambda b: (b, 0)),
            pl.BlockSpec((ROWS_BLK, DH), lambda b: (b, 0)),
            pl.BlockSpec((ROWS_BLK, D), lambda b: (b, 0)),
            pl.BlockSpec((D, D), lambda b: (0, 0)),
            pl.BlockSpec((D, D), lambda b: (0, 0)),
        ],
        out_specs=pl.BlockSpec((ROWS_BLK, D), lambda b: (b, 0)),
        out_shape=jax.ShapeDtypeStruct((N_NODES, D), jnp.float32),
    )(m0, m1, x_dst, W_l, W_r)


def kernel(x_user, x_item, edge_index_ui, edge_index_iu, W_l_ui, W_r_ui, W_l_iu, W_r_iu):
    xu0 = x_user[:, :DH]
    xu1 = x_user[:, DH:]
    xi0 = x_item[:, :DH]
    xi1 = x_item[:, DH:]
    pad = jnp.zeros((16,), jnp.int32)
    sui = jnp.concatenate([edge_index_ui[0].astype(jnp.int32), pad])
    dui = jnp.concatenate([edge_index_ui[1].astype(jnp.int32), pad])
    siu = jnp.concatenate([edge_index_iu[0].astype(jnp.int32), pad])
    diu = jnp.concatenate([edge_index_iu[1].astype(jnp.int32), pad])

    mean_i0, mean_i1, mean_u0, mean_u1 = _sc_segment_means(
        xu0, xu1, xi0, xi1, sui, dui, siu, diu)

    out_item = _tc_sage_update(mean_i0, mean_i1, x_item, W_l_ui, W_r_ui)
    out_user = _tc_sage_update(mean_u0, mean_u1, x_user, W_l_iu, W_r_iu)
    return (out_user, out_item)


# speedup vs baseline: 1.5712x; 1.1063x over previous
import functools

import jax
import jax.numpy as jnp
from jax import lax
from jax.experimental import pallas as pl
from jax.experimental.pallas import tpu as pltpu
from jax.experimental.pallas import tpu_sc as plsc

N_NODES = 10000
NP = 10240
D = 256
DH = 128
E = 160000

NS = 16
RPT = NP // NS
EB = 8000
NB = E // EB
GC = 128
SCR = 64
CNTW = 16
HPT = RPT // 2
RA = 256
RB = HPT - RA


def _sc_segment_means(xu0, xu1, xi0, xi1, sui, dui, siu, diu):
    mesh = plsc.VectorSubcoreMesh(core_axis_name="c", subcore_axis_name="s")
    f32 = jnp.float32
    i32 = jnp.int32

    @functools.partial(
        pl.kernel,
        out_type=[
            jax.ShapeDtypeStruct((NP, DH), f32),
            jax.ShapeDtypeStruct((NP, DH), f32),
            jax.ShapeDtypeStruct((NP, DH), f32),
            jax.ShapeDtypeStruct((NP, DH), f32),
        ],
        mesh=mesh,
        compiler_params=pltpu.CompilerParams(needs_layout_passes=False),
    )
    def k(xu0_h, xu1_h, xi0_h, xi1_h, sui_h, dui_h, siu_h, diu_h,
          mean_i0, mean_i1, mean_u0, mean_u1):
        pl.run_scoped(
            functools.partial(
                _tile_body, xu0_h, xu1_h, xi0_h, xi1_h,
                sui_h, dui_h, siu_h, diu_h,
                mean_i0, mean_i1, mean_u0, mean_u1),
            pltpu.VMEM((RA, DH), f32),
            pltpu.VMEM((RB, DH), f32),
            pltpu.VMEM((RA, CNTW), f32),
            pltpu.VMEM((RB, CNTW), f32),
            pltpu.VMEM((GC, DH), f32),
            pltpu.VMEM((EB + 16,), i32),
            pltpu.VMEM((EB + 16,), i32),
            pltpu.VMEM((GC,), i32),
        )

    def _tile_body(xu0_h, xu1_h, xi0_h, xi1_h, sui_h, dui_h, siu_h, diu_h,
                   mean_i0, mean_i1, mean_u0, mean_u1,
                   acc_a, acc_b, cnt_a, cnt_b, stage, sstage, dstage, gidx):
        c = lax.axis_index("c")
        s = lax.axis_index("s")
        row0 = s * RPT

        one16 = jnp.full((16,), 1.0, f32)
        zero16 = jnp.zeros((16,), f32)
        zi16 = jnp.zeros((16,), i32)

        for rel in range(2):
          x0_h, x1_h = (xu0_h, xu1_h) if rel == 0 else (xi0_h, xi1_h)
          si_h, di_h = (sui_h, dui_h) if rel == 0 else (siu_h, diu_h)
          o0, o1 = (mean_i0, mean_i1) if rel == 0 else (mean_u0, mean_u1)
          for half in range(2):
            rowb = row0 + half * HPT

            def zero_row_a(r, _):
                for kk in range(DH // 16):
                    acc_a[r, pl.ds(kk * 16, 16)] = zero16
                cnt_a[r] = zero16
                return 0

            def zero_row_b(r, _):
                for kk in range(DH // 16):
                    acc_b[r, pl.ds(kk * 16, 16)] = zero16
                cnt_b[r] = zero16
                return 0

            lax.fori_loop(0, RA, zero_row_a, 0)
            lax.fori_loop(0, RB, zero_row_b, 0)

            def block_body(b, _):
                pltpu.sync_copy(si_h.at[pl.ds(b * EB, EB + 16)], sstage)
                pltpu.sync_copy(di_h.at[pl.ds(b * EB, EB + 16)], dstage)

                def scan_step(v, p):
                    d16 = dstage[pl.ds(v * 16, 16)]
                    s16 = sstage[pl.ds(v * 16, 16)]
                    m = (d16 - rowb).astype(jnp.uint32) < jnp.uint32(HPT)
                    mi = jnp.where(m, jnp.int32(1), jnp.int32(0))
                    pos = plsc.cumsum(mi)
                    idx = jnp.where(m, p + pos - 1, jnp.int32(EB))
                    plsc.store_scatter(sstage, [idx], s16)
                    plsc.store_scatter(dstage, [idx], d16)
                    return p + pos[15]

                p = lax.fori_loop(0, EB // 16, scan_step, jnp.int32(0))

                def chunk_body(g, _):
                    for kk in range(GC // 16):
                        gidx[pl.ds(kk * 16, 16)] = (
                            sstage[pl.ds(g * GC + kk * 16, 16)])

                    @pl.when(c == 0)
                    def _():
                        pltpu.sync_copy(x0_h.at[gidx], stage)

                    @pl.when(c == 1)
                    def _():
                        pltpu.sync_copy(x1_h.at[gidx], stage)

                    n = jnp.minimum(p - g * GC, GC)

                    def edge_body(e, _):
                        dloc = dstage[pl.ds(g * GC + e, 16)][0] - rowb

                        @pl.when(dloc < RA)
                        def _():
                            for kk in range(DH // 16):
                                plsc.addupdate(
                                    acc_a.at[dloc, pl.ds(kk * 16, 16)],
                                    stage[e, pl.ds(kk * 16, 16)])
                            plsc.addupdate(cnt_a.at[dloc], one16)

                        @pl.when(dloc >= RA)
                        def _():
                            for kk in range(DH // 16):
                                plsc.addupdate(
                                    acc_b.at[dloc - RA, pl.ds(kk * 16, 16)],
                                    stage[e, pl.ds(kk * 16, 16)])
                            plsc.addupdate(cnt_b.at[dloc - RA], one16)
                        return 0

                    lax.fori_loop(0, n, edge_body, 0)
                    return 0

                lax.fori_loop(0, (p + GC - 1) // GC, chunk_body, 0)
                return 0

            lax.fori_loop(0, NB, block_body, 0)

            for cc in range(HPT // SCR):
                part_acc = acc_a if cc * SCR < RA else acc_b
                part_cnt = cnt_a if cc * SCR < RA else cnt_b
                rbase = cc * SCR if cc * SCR < RA else cc * SCR - RA

                def scale_row(r, _, part_acc=part_acc, part_cnt=part_cnt,
                              rbase=rbase):
                    rr = rbase + r
                    c16 = part_cnt[rr]
                    inv = 1.0 / jnp.maximum(c16, 1.0)
                    for kk in range(DH // 16):
                        stage[r, pl.ds(kk * 16, 16)] = (
                            part_acc[rr, pl.ds(kk * 16, 16)] * inv)
                    return 0

                lax.fori_loop(0, SCR, scale_row, 0)
                orow = rowb + cc * SCR

                @pl.when(c == 0)
                def _():
                    pltpu.sync_copy(stage.at[pl.ds(0, SCR)],
                                    o0.at[pl.ds(orow, SCR)])

                @pl.when(c == 1)
                def _():
                    pltpu.sync_copy(stage.at[pl.ds(0, SCR)],
                                    o1.at[pl.ds(orow, SCR)])

        return None

    _ = _tile_body
    return k(xu0, xu1, xi0, xi1, sui, dui, siu, diu)


ROWS_BLK = 400


def _tc_body(m0_ref, m1_ref, xd_ref, wl_ref, wr_ref, out_ref):
    out_ref[...] = (
        jnp.dot(m0_ref[...], wl_ref[0:DH, :], preferred_element_type=jnp.float32)
        + jnp.dot(m1_ref[...], wl_ref[DH:D, :], preferred_element_type=jnp.float32)
        + jnp.dot(xd_ref[...], wr_ref[...], preferred_element_type=jnp.float32)
    )


def _tc_sage_update(m0, m1, x_dst, W_l, W_r):
    grid = (N_NODES // ROWS_BLK,)
    return pl.pallas_call(
        _tc_body,
        grid=grid,
        in_specs=[
            pl.BlockSpec((ROWS_BLK, DH), lambda b: (b, 0)),
            pl.BlockSpec((ROWS_BLK, DH), lambda b: (b, 0)),
            pl.BlockSpec((ROWS_BLK, D), lambda b: (b, 0)),
            pl.BlockSpec((D, D), lambda b: (0, 0)),
            pl.BlockSpec((D, D), lambda b: (0, 0)),
        ],
        out_specs=pl.BlockSpec((ROWS_BLK, D), lambda b: (b, 0)),
        out_shape=jax.ShapeDtypeStruct((N_NODES, D), jnp.float32),
    )(m0, m1, x_dst, W_l, W_r)


def kernel(x_user, x_item, edge_index_ui, edge_index_iu, W_l_ui, W_r_ui, W_l_iu, W_r_iu):
    xu0 = x_user[:, :DH]
    xu1 = x_user[:, DH:]
    xi0 = x_item[:, :DH]
    xi1 = x_item[:, DH:]
    pad = jnp.zeros((16,), jnp.int32)
    sui = jnp.concatenate([edge_index_ui[0].astype(jnp.int32), pad])
    dui = jnp.concatenate([edge_index_ui[1].astype(jnp.int32), pad])
    siu = jnp.concatenate([edge_index_iu[0].astype(jnp.int32), pad])
    diu = jnp.concatenate([edge_index_iu[1].astype(jnp.int32), pad])

    mean_i0, mean_i1, mean_u0, mean_u1 = _sc_segment_means(
        xu0, xu1, xi0, xi1, sui, dui, siu, diu)

    out_item = _tc_sage_update(mean_i0, mean_i1, x_item, W_l_ui, W_r_ui)
    out_user = _tc_sage_update(mean_u0, mean_u1, x_user, W_l_iu, W_r_iu)
    return (out_user, out_item)


# speedup vs baseline: 1.6499x; 1.0501x over previous
import functools

import jax
import jax.numpy as jnp
from jax import lax
from jax.experimental import pallas as pl
from jax.experimental.pallas import tpu as pltpu
from jax.experimental.pallas import tpu_sc as plsc

N_NODES = 10000
NP = 10240
D = 256
DH = 128
E = 160000

NS = 16
RPT = NP // NS
EB = 16000
NB = E // EB
GC = 128
SCR = 64
CNTW = 16
HPT = RPT // 2
RA = 256
RB = HPT - RA


def _sc_segment_means(xu0, xu1, xi0, xi1, sui, dui, siu, diu):
    mesh = plsc.VectorSubcoreMesh(core_axis_name="c", subcore_axis_name="s")
    f32 = jnp.float32
    i32 = jnp.int32

    @functools.partial(
        pl.kernel,
        out_type=[
            jax.ShapeDtypeStruct((NP, DH), f32),
            jax.ShapeDtypeStruct((NP, DH), f32),
            jax.ShapeDtypeStruct((NP, DH), f32),
            jax.ShapeDtypeStruct((NP, DH), f32),
        ],
        mesh=mesh,
        compiler_params=pltpu.CompilerParams(needs_layout_passes=False),
    )
    def k(xu0_h, xu1_h, xi0_h, xi1_h, sui_h, dui_h, siu_h, diu_h,
          mean_i0, mean_i1, mean_u0, mean_u1):
        pl.run_scoped(
            functools.partial(
                _tile_body, xu0_h, xu1_h, xi0_h, xi1_h,
                sui_h, dui_h, siu_h, diu_h,
                mean_i0, mean_i1, mean_u0, mean_u1),
            pltpu.VMEM((RA, DH), f32),
            pltpu.VMEM((RB, DH), f32),
            pltpu.VMEM((RA, CNTW), f32),
            pltpu.VMEM((RB, CNTW), f32),
            pltpu.VMEM((GC, DH), f32),
            pltpu.VMEM((EB + 16,), i32),
            pltpu.VMEM((EB + 16,), i32),
            pltpu.VMEM((GC,), i32),
        )

    def _tile_body(xu0_h, xu1_h, xi0_h, xi1_h, sui_h, dui_h, siu_h, diu_h,
                   mean_i0, mean_i1, mean_u0, mean_u1,
                   acc_a, acc_b, cnt_a, cnt_b, stage, sstage, dstage, gidx):
        c = lax.axis_index("c")
        s = lax.axis_index("s")
        row0 = s * RPT

        one16 = jnp.full((16,), 1.0, f32)
        zero16 = jnp.zeros((16,), f32)
        zi16 = jnp.zeros((16,), i32)

        for rel in range(2):
          x0_h, x1_h = (xu0_h, xu1_h) if rel == 0 else (xi0_h, xi1_h)
          si_h, di_h = (sui_h, dui_h) if rel == 0 else (siu_h, diu_h)
          o0, o1 = (mean_i0, mean_i1) if rel == 0 else (mean_u0, mean_u1)
          for half in range(2):
            rowb = row0 + half * HPT

            def zero_row_a(r, _):
                for kk in range(DH // 16):
                    acc_a[r, pl.ds(kk * 16, 16)] = zero16
                cnt_a[r] = zero16
                return 0

            def zero_row_b(r, _):
                for kk in range(DH // 16):
                    acc_b[r, pl.ds(kk * 16, 16)] = zero16
                cnt_b[r] = zero16
                return 0

            lax.fori_loop(0, RA, zero_row_a, 0)
            lax.fori_loop(0, RB, zero_row_b, 0)

            def block_body(b, _):
                pltpu.sync_copy(si_h.at[pl.ds(b * EB, EB + 16)], sstage)
                pltpu.sync_copy(di_h.at[pl.ds(b * EB, EB + 16)], dstage)

                def scan_step(v, p):
                    d16 = dstage[pl.ds(v * 16, 16)]
                    s16 = sstage[pl.ds(v * 16, 16)]
                    m = (d16 - rowb).astype(jnp.uint32) < jnp.uint32(HPT)
                    mi = jnp.where(m, jnp.int32(1), jnp.int32(0))
                    pos = plsc.cumsum(mi)
                    idx = jnp.where(m, p + pos - 1, jnp.int32(EB))
                    plsc.store_scatter(sstage, [idx], s16)
                    plsc.store_scatter(dstage, [idx], d16)
                    return p + pos[15]

                p = lax.fori_loop(0, EB // 16, scan_step, jnp.int32(0))

                def chunk_body(g, _):
                    for kk in range(GC // 16):
                        gidx[pl.ds(kk * 16, 16)] = (
                            sstage[pl.ds(g * GC + kk * 16, 16)])

                    @pl.when(c == 0)
                    def _():
                        pltpu.sync_copy(x0_h.at[gidx], stage)

                    @pl.when(c == 1)
                    def _():
                        pltpu.sync_copy(x1_h.at[gidx], stage)

                    n = jnp.minimum(p - g * GC, GC)

                    def edge_body(e, _):
                        dloc = dstage[pl.ds(g * GC + e, 16)][0] - rowb

                        @pl.when(dloc < RA)
                        def _():
                            for kk in range(DH // 16):
                                plsc.addupdate(
                                    acc_a.at[dloc, pl.ds(kk * 16, 16)],
                                    stage[e, pl.ds(kk * 16, 16)])
                            plsc.addupdate(cnt_a.at[dloc], one16)

                        @pl.when(dloc >= RA)
                        def _():
                            for kk in range(DH // 16):
                                plsc.addupdate(
                                    acc_b.at[dloc - RA, pl.ds(kk * 16, 16)],
                                    stage[e, pl.ds(kk * 16, 16)])
                            plsc.addupdate(cnt_b.at[dloc - RA], one16)
                        return 0

                    lax.fori_loop(0, n, edge_body, 0)
                    return 0

                lax.fori_loop(0, (p + GC - 1) // GC, chunk_body, 0)
                return 0

            lax.fori_loop(0, NB, block_body, 0)

            for cc in range(HPT // SCR):
                part_acc = acc_a if cc * SCR < RA else acc_b
                part_cnt = cnt_a if cc * SCR < RA else cnt_b
                rbase = cc * SCR if cc * SCR < RA else cc * SCR - RA

                def scale_row(r, _, part_acc=part_acc, part_cnt=part_cnt,
                              rbase=rbase):
                    rr = rbase + r
                    c16 = part_cnt[rr]
                    inv = 1.0 / jnp.maximum(c16, 1.0)
                    for kk in range(DH // 16):
                        stage[r, pl.ds(kk * 16, 16)] = (
                            part_acc[rr, pl.ds(kk * 16, 16)] * inv)
                    return 0

                lax.fori_loop(0, SCR, scale_row, 0)
                orow = rowb + cc * SCR

                @pl.when(c == 0)
                def _():
                    pltpu.sync_copy(stage.at[pl.ds(0, SCR)],
                                    o0.at[pl.ds(orow, SCR)])

                @pl.when(c == 1)
                def _():
                    pltpu.sync_copy(stage.at[pl.ds(0, SCR)],
                                    o1.at[pl.ds(orow, SCR)])

        return None

    _ = _tile_body
    return k(xu0, xu1, xi0, xi1, sui, dui, siu, diu)


ROWS_BLK = 400


def _tc_body(m0_ref, m1_ref, xd_ref, wl_ref, wr_ref, out_ref):
    out_ref[...] = (
        jnp.dot(m0_ref[...], wl_ref[0:DH, :], preferred_element_type=jnp.float32)
        + jnp.dot(m1_ref[...], wl_ref[DH:D, :], preferred_element_type=jnp.float32)
        + jnp.dot(xd_ref[...], wr_ref[...], preferred_element_type=jnp.float32)
    )


def _tc_sage_update(m0, m1, x_dst, W_l, W_r):
    grid = (N_NODES // ROWS_BLK,)
    return pl.pallas_call(
        _tc_body,
        grid=grid,
        in_specs=[
            pl.BlockSpec((ROWS_BLK, DH), lambda b: (b, 0)),
            pl.BlockSpec((ROWS_BLK, DH), lambda b: (b, 0)),
            pl.BlockSpec((ROWS_BLK, D), lambda b: (b, 0)),
            pl.BlockSpec((D, D), lambda b: (0, 0)),
            pl.BlockSpec((D, D), lambda b: (0, 0)),
        ],
        out_specs=pl.BlockSpec((ROWS_BLK, D), lambda b: (b, 0)),
        out_shape=jax.ShapeDtypeStruct((N_NODES, D), jnp.float32),
    )(m0, m1, x_dst, W_l, W_r)


def kernel(x_user, x_item, edge_index_ui, edge_index_iu, W_l_ui, W_r_ui, W_l_iu, W_r_iu):
    xu0 = x_user[:, :DH]
    xu1 = x_user[:, DH:]
    xi0 = x_item[:, :DH]
    xi1 = x_item[:, DH:]
    pad = jnp.zeros((16,), jnp.int32)
    sui = jnp.concatenate([edge_index_ui[0].astype(jnp.int32), pad])
    dui = jnp.concatenate([edge_index_ui[1].astype(jnp.int32), pad])
    siu = jnp.concatenate([edge_index_iu[0].astype(jnp.int32), pad])
    diu = jnp.concatenate([edge_index_iu[1].astype(jnp.int32), pad])

    mean_i0, mean_i1, mean_u0, mean_u1 = _sc_segment_means(
        xu0, xu1, xi0, xi1, sui, dui, siu, diu)

    out_item = _tc_sage_update(mean_i0, mean_i1, x_item, W_l_ui, W_r_ui)
    out_user = _tc_sage_update(mean_u0, mean_u1, x_user, W_l_iu, W_r_iu)
    return (out_user, out_item)


# speedup vs baseline: 1.6917x; 1.0253x over previous
import functools

import jax
import jax.numpy as jnp
from jax import lax
from jax.experimental import pallas as pl
from jax.experimental.pallas import tpu as pltpu
from jax.experimental.pallas import tpu_sc as plsc

N_NODES = 10000
NP = 10240
D = 256
DH = 128
E = 160000

NS = 16
RPT = NP // NS
EB = 16000
NB = E // EB
GC = 128
SCR = 64
CNTW = 16
HPT = RPT // 2
RA = 256
RB = HPT - RA


def _sc_segment_means(xu0, xu1, xi0, xi1, sui, dui, siu, diu):
    mesh = plsc.VectorSubcoreMesh(core_axis_name="c", subcore_axis_name="s")
    f32 = jnp.float32
    i32 = jnp.int32

    @functools.partial(
        pl.kernel,
        out_type=[
            jax.ShapeDtypeStruct((NP, DH), f32),
            jax.ShapeDtypeStruct((NP, DH), f32),
            jax.ShapeDtypeStruct((NP, DH), f32),
            jax.ShapeDtypeStruct((NP, DH), f32),
        ],
        mesh=mesh,
        compiler_params=pltpu.CompilerParams(needs_layout_passes=False),
    )
    def k(xu0_h, xu1_h, xi0_h, xi1_h, sui_h, dui_h, siu_h, diu_h,
          mean_i0, mean_i1, mean_u0, mean_u1):
        pl.run_scoped(
            functools.partial(
                _tile_body, xu0_h, xu1_h, xi0_h, xi1_h,
                sui_h, dui_h, siu_h, diu_h,
                mean_i0, mean_i1, mean_u0, mean_u1),
            pltpu.VMEM((RA, DH), f32),
            pltpu.VMEM((RB, DH), f32),
            pltpu.VMEM((RA, CNTW), f32),
            pltpu.VMEM((RB, CNTW), f32),
            pltpu.VMEM((GC, DH), f32),
            pltpu.VMEM((EB + 16,), i32),
            pltpu.VMEM((EB + 16,), i32),
            pltpu.VMEM((GC,), i32),
        )

    def _tile_body(xu0_h, xu1_h, xi0_h, xi1_h, sui_h, dui_h, siu_h, diu_h,
                   mean_i0, mean_i1, mean_u0, mean_u1,
                   acc_a, acc_b, cnt_a, cnt_b, stage, sstage, dstage, gidx):
        c = lax.axis_index("c")
        s = lax.axis_index("s")
        row0 = s * RPT

        one16 = jnp.full((16,), 1.0, f32)
        zero16 = jnp.zeros((16,), f32)
        zi16 = jnp.zeros((16,), i32)

        for rel in range(2):
          x0_h, x1_h = (xu0_h, xu1_h) if rel == 0 else (xi0_h, xi1_h)
          si_h, di_h = (sui_h, dui_h) if rel == 0 else (siu_h, diu_h)
          o0, o1 = (mean_i0, mean_i1) if rel == 0 else (mean_u0, mean_u1)
          for half in range(2):
            rowb = row0 + half * HPT

            def zero_row_a(r, _):
                for kk in range(DH // 16):
                    acc_a[r, pl.ds(kk * 16, 16)] = zero16
                cnt_a[r] = zero16
                return 0

            def zero_row_b(r, _):
                for kk in range(DH // 16):
                    acc_b[r, pl.ds(kk * 16, 16)] = zero16
                cnt_b[r] = zero16
                return 0

            lax.fori_loop(0, RA, zero_row_a, 0)
            lax.fori_loop(0, RB, zero_row_b, 0)

            def block_body(b, _):
                pltpu.sync_copy(si_h.at[pl.ds(b * EB, EB + 16)], sstage)
                pltpu.sync_copy(di_h.at[pl.ds(b * EB, EB + 16)], dstage)

                def scan_step(v, p):
                    for u in range(4):
                        off = v * 64 + u * 16
                        d16 = dstage[pl.ds(off, 16)]
                        s16 = sstage[pl.ds(off, 16)]
                        m = (d16 - rowb).astype(jnp.uint32) < jnp.uint32(HPT)
                        pos = plsc.cumsum(m.astype(jnp.int32))
                        idx = jnp.where(m, p + pos - 1, jnp.int32(EB))
                        plsc.store_scatter(sstage, [idx], s16)
                        plsc.store_scatter(dstage, [idx], d16)
                        p = p + pos[15]
                    return p

                p = lax.fori_loop(0, EB // 64, scan_step, jnp.int32(0))

                def chunk_body(g, _):
                    for kk in range(GC // 16):
                        gidx[pl.ds(kk * 16, 16)] = (
                            sstage[pl.ds(g * GC + kk * 16, 16)])

                    @pl.when(c == 0)
                    def _():
                        pltpu.sync_copy(x0_h.at[gidx], stage)

                    @pl.when(c == 1)
                    def _():
                        pltpu.sync_copy(x1_h.at[gidx], stage)

                    n = jnp.minimum(p - g * GC, GC)

                    def edge_body(e, _):
                        dloc = dstage[pl.ds(g * GC + e, 16)][0] - rowb

                        @pl.when(dloc < RA)
                        def _():
                            for kk in range(DH // 16):
                                plsc.addupdate(
                                    acc_a.at[dloc, pl.ds(kk * 16, 16)],
                                    stage[e, pl.ds(kk * 16, 16)])
                            plsc.addupdate(cnt_a.at[dloc], one16)

                        @pl.when(dloc >= RA)
                        def _():
                            for kk in range(DH // 16):
                                plsc.addupdate(
                                    acc_b.at[dloc - RA, pl.ds(kk * 16, 16)],
                                    stage[e, pl.ds(kk * 16, 16)])
                            plsc.addupdate(cnt_b.at[dloc - RA], one16)
                        return 0

                    lax.fori_loop(0, n, edge_body, 0)
                    return 0

                lax.fori_loop(0, (p + GC - 1) // GC, chunk_body, 0)
                return 0

            lax.fori_loop(0, NB, block_body, 0)

            for cc in range(HPT // SCR):
                part_acc = acc_a if cc * SCR < RA else acc_b
                part_cnt = cnt_a if cc * SCR < RA else cnt_b
                rbase = cc * SCR if cc * SCR < RA else cc * SCR - RA

                def scale_row(r, _, part_acc=part_acc, part_cnt=part_cnt,
                              rbase=rbase):
                    rr = rbase + r
                    c16 = part_cnt[rr]
                    inv = 1.0 / jnp.maximum(c16, 1.0)
                    for kk in range(DH // 16):
                        stage[r, pl.ds(kk * 16, 16)] = (
                            part_acc[rr, pl.ds(kk * 16, 16)] * inv)
                    return 0

                lax.fori_loop(0, SCR, scale_row, 0)
                orow = rowb + cc * SCR

                @pl.when(c == 0)
                def _():
                    pltpu.sync_copy(stage.at[pl.ds(0, SCR)],
                                    o0.at[pl.ds(orow, SCR)])

                @pl.when(c == 1)
                def _():
                    pltpu.sync_copy(stage.at[pl.ds(0, SCR)],
                                    o1.at[pl.ds(orow, SCR)])

        return None

    _ = _tile_body
    return k(xu0, xu1, xi0, xi1, sui, dui, siu, diu)


ROWS_BLK = 400


def _tc_body(m0_ref, m1_ref, xd_ref, wl_ref, wr_ref, out_ref):
    out_ref[...] = (
        jnp.dot(m0_ref[...], wl_ref[0:DH, :], preferred_element_type=jnp.float32)
        + jnp.dot(m1_ref[...], wl_ref[DH:D, :], preferred_element_type=jnp.float32)
        + jnp.dot(xd_ref[...], wr_ref[...], preferred_element_type=jnp.float32)
    )


def _tc_sage_update(m0, m1, x_dst, W_l, W_r):
    grid = (N_NODES // ROWS_BLK,)
    return pl.pallas_call(
        _tc_body,
        grid=grid,
        in_specs=[
            pl.BlockSpec((ROWS_BLK, DH), lambda b: (b, 0)),
            pl.BlockSpec((ROWS_BLK, DH), lambda b: (b, 0)),
            pl.BlockSpec((ROWS_BLK, D), lambda b: (b, 0)),
            pl.BlockSpec((D, D), lambda b: (0, 0)),
            pl.BlockSpec((D, D), lambda b: (0, 0)),
        ],
        out_specs=pl.BlockSpec((ROWS_BLK, D), lambda b: (b, 0)),
        out_shape=jax.ShapeDtypeStruct((N_NODES, D), jnp.float32),
    )(m0, m1, x_dst, W_l, W_r)


def kernel(x_user, x_item, edge_index_ui, edge_index_iu, W_l_ui, W_r_ui, W_l_iu, W_r_iu):
    xu0 = x_user[:, :DH]
    xu1 = x_user[:, DH:]
    xi0 = x_item[:, :DH]
    xi1 = x_item[:, DH:]
    pad = jnp.zeros((16,), jnp.int32)
    sui = jnp.concatenate([edge_index_ui[0].astype(jnp.int32), pad])
    dui = jnp.concatenate([edge_index_ui[1].astype(jnp.int32), pad])
    siu = jnp.concatenate([edge_index_iu[0].astype(jnp.int32), pad])
    diu = jnp.concatenate([edge_index_iu[1].astype(jnp.int32), pad])

    mean_i0, mean_i1, mean_u0, mean_u1 = _sc_segment_means(
        xu0, xu1, xi0, xi1, sui, dui, siu, diu)

    out_item = _tc_sage_update(mean_i0, mean_i1, x_item, W_l_ui, W_r_ui)
    out_user = _tc_sage_update(mean_u0, mean_u1, x_user, W_l_iu, W_r_iu)
    return (out_user, out_item)
